# baseline (device time: 96642 ns/iter reference)
import jax
import jax.numpy as jnp
from jax import lax
from jax.experimental import pallas as pl
from jax.experimental.pallas import tpu as pltpu

N_DEV = 8
N_RING = 4
NSUB = 2


def kernel(x, w_mat, scale_x, scale_w):
    m, k = x.shape
    _, n = w_mat.shape
    chunk = m // N_DEV
    sc = 2 * chunk
    nh = n // 2
    sub = nh // NSUB

    def body(x_ref, w_ref, sx_ref, sw_ref, out_ref,
             x8, w8, comm_cw, comm_ccw, zrecv_cw, zrecv_ccw,
             send_cw, recv_cw, send_ccw, recv_ccw,
             zsend_cw, zrecv_sem_cw, zsend_ccw, zrecv_sem_ccw):
        p = lax.axis_index("i")
        g = p % N_RING
        base = p - g
        right = base + (g + 1) % N_RING
        left = base + (g + N_RING - 1) % N_RING
        zp = (p + N_RING) % N_DEV
        my_off = jnp.where(p < N_RING, 0, chunk)
        pa_off = chunk - my_off

        bar = pltpu.get_barrier_semaphore()
        for nbr in (left, right, zp):
            pl.semaphore_signal(bar, inc=1, device_id=(nbr,),
                                device_id_type=pl.DeviceIdType.MESH)
        pl.semaphore_wait(bar, 3)

        x8[...] = x_ref[...].astype(x8.dtype)
        w8[...] = w_ref[...].astype(w8.dtype)

        def spartial(c, hi):
            b = w8[:, nh:] if hi else w8[:, :nh]
            t = jnp.dot(x8[pl.ds(c * chunk, chunk), :], b,
                        preferred_element_type=jnp.float32)
            bt = jnp.dot(x8[pl.ds((c + N_RING) * chunk, chunk), :], b,
                         preferred_element_type=jnp.float32)
            return t, bt

        def make_rdma(comm, sems_s, sems_r, h, j, tgt):
            cs = j * sub
            return pltpu.make_async_remote_copy(
                src_ref=comm.at[h, :, pl.ds(cs, sub)],
                dst_ref=comm.at[h + 1, :, pl.ds(cs, sub)],
                send_sem=sems_s.at[h, j], recv_sem=sems_r.at[h + 1, j],
                device_id=(tgt,), device_id_type=pl.DeviceIdType.MESH)

        def accum(comm, h, j, top, bot):
            cs = j * sub
            comm[h, 0:chunk, pl.ds(cs, sub)] = (
                comm[h, 0:chunk, pl.ds(cs, sub)].astype(jnp.float32)
                + top[:, cs:cs + sub]).astype(jnp.bfloat16)
            comm[h, chunk:sc, pl.ds(cs, sub)] = (
                comm[h, chunk:sc, pl.ds(cs, sub)].astype(jnp.float32)
                + bot[:, cs:cs + sub]).astype(jnp.bfloat16)

        t, bt = spartial((g + N_RING - 1) % N_RING, hi=False)
        comm_cw[0, 0:chunk, :] = t.astype(jnp.bfloat16)
        comm_cw[0, chunk:sc, :] = bt.astype(jnp.bfloat16)
        t, bt = spartial((g + 1) % N_RING, hi=True)
        comm_ccw[0, 0:chunk, :] = t.astype(jnp.bfloat16)
        comm_ccw[0, chunk:sc, :] = bt.astype(jnp.bfloat16)
        prev = []
        for j in range(NSUB):
            rd_cw = make_rdma(comm_cw, send_cw, recv_cw, 0, j, right)
            rd_ccw = make_rdma(comm_ccw, send_ccw, recv_ccw, 0, j, left)
            rd_cw.start()
            rd_ccw.start()
            prev.append((rd_cw, rd_ccw))

        for h in range(1, N_RING - 1):
            tcw, bcw = spartial((g + N_RING - 1 - h) % N_RING, hi=False)
            tccw, bccw = spartial((g + 1 + h) % N_RING, hi=True)
            cur = []
            for j in range(NSUB):
                rd_cw, rd_ccw = prev[j]
                rd_cw.wait()
                accum(comm_cw, h, j, tcw, bcw)
                nrd_cw = make_rdma(comm_cw, send_cw, recv_cw, h, j, right)
                nrd_cw.start()
                rd_ccw.wait()
                accum(comm_ccw, h, j, tccw, bccw)
                nrd_ccw = make_rdma(comm_ccw, send_ccw, recv_ccw, h, j, left)
                nrd_ccw.start()
                cur.append((nrd_cw, nrd_ccw))
            prev = cur

        last = N_RING - 1
        tcw, bcw = spartial(g, hi=False)
        tccw, bccw = spartial(g, hi=True)
        zs = []
        for j in range(NSUB):
            cs = j * sub
            rd_cw, rd_ccw = prev[j]
            rd_cw.wait()
            accum(comm_cw, last, j, tcw, bcw)
            z_cw = pltpu.make_async_remote_copy(
                src_ref=comm_cw.at[last, pl.ds(pa_off, chunk), pl.ds(cs, sub)],
                dst_ref=zrecv_cw.at[:, pl.ds(cs, sub)],
                send_sem=zsend_cw.at[j], recv_sem=zrecv_sem_cw.at[j],
                device_id=(zp,), device_id_type=pl.DeviceIdType.MESH)
            z_cw.start()
            rd_ccw.wait()
            accum(comm_ccw, last, j, tccw, bccw)
            z_ccw = pltpu.make_async_remote_copy(
                src_ref=comm_ccw.at[last, pl.ds(pa_off, chunk), pl.ds(cs, sub)],
                dst_ref=zrecv_ccw.at[:, pl.ds(cs, sub)],
                send_sem=zsend_ccw.at[j], recv_sem=zrecv_sem_ccw.at[j],
                device_id=(zp,), device_id_type=pl.DeviceIdType.MESH)
            z_ccw.start()
            zs.append((z_cw, z_ccw))

        s = sx_ref[0] * sw_ref[0]
        for z_cw, z_ccw in zs:
            z_cw.wait()
            z_ccw.wait()
        acc_cw = (comm_cw[last, pl.ds(my_off, chunk), :].astype(jnp.float32)
                  + zrecv_cw[...].astype(jnp.float32))
        acc_ccw = (comm_ccw[last, pl.ds(my_off, chunk), :].astype(jnp.float32)
                   + zrecv_ccw[...].astype(jnp.float32))
        out_ref[:, :nh] = jnp.maximum(acc_cw * s, 0.0)
        out_ref[:, nh:] = jnp.maximum(acc_ccw * s, 0.0)

    return pl.pallas_call(
        body,
        out_shape=jax.ShapeDtypeStruct((chunk, n), jnp.float32),
        in_specs=[
            pl.BlockSpec(memory_space=pltpu.VMEM),
            pl.BlockSpec(memory_space=pltpu.VMEM),
            pl.BlockSpec(memory_space=pltpu.SMEM),
            pl.BlockSpec(memory_space=pltpu.SMEM),
        ],
        out_specs=pl.BlockSpec(memory_space=pltpu.VMEM),
        scratch_shapes=[
            pltpu.VMEM((m, k), jnp.float8_e4m3fn),
            pltpu.VMEM((k, n), jnp.float8_e5m2),
            pltpu.VMEM((N_RING, sc, nh), jnp.bfloat16),
            pltpu.VMEM((N_RING, sc, nh), jnp.bfloat16),
            pltpu.VMEM((chunk, nh), jnp.bfloat16),
            pltpu.VMEM((chunk, nh), jnp.bfloat16),
            pltpu.SemaphoreType.DMA((N_RING, NSUB)),
            pltpu.SemaphoreType.DMA((N_RING, NSUB)),
            pltpu.SemaphoreType.DMA((N_RING, NSUB)),
            pltpu.SemaphoreType.DMA((N_RING, NSUB)),
            pltpu.SemaphoreType.DMA((NSUB,)),
            pltpu.SemaphoreType.DMA((NSUB,)),
            pltpu.SemaphoreType.DMA((NSUB,)),
            pltpu.SemaphoreType.DMA((NSUB,)),
        ],
        compiler_params=pltpu.CompilerParams(collective_id=0),
    )(x, w_mat, scale_x, scale_w)


# device time: 70877 ns/iter; 1.3635x vs baseline; 1.3635x over previous
import jax
import jax.numpy as jnp
from jax import lax
from jax.experimental import pallas as pl
from jax.experimental.pallas import tpu as pltpu

N_DEV = 8
N_RING = 4
CH = 512
AW = 512
BW = 512
ASUB = 2
ASW = AW // ASUB


def kernel(x, w_mat, scale_x, scale_w):
    m, k = x.shape
    _, n = w_mat.shape
    f32 = jnp.float32
    bf16 = jnp.bfloat16

    def body(x_ref, w_ref, sx_ref, sw_ref, out_ref,
             x8, w8,
             comm_a_cw, comm_a_ccw, za_cw, za_ccw,
             bsend, bmine, zbr, comm_b_cw, comm_b_ccw,
             a_send_cw, a_recv_cw, a_send_ccw, a_recv_ccw,
             az_send_cw, az_recv_cw, az_send_ccw, az_recv_ccw,
             bz_send, bz_recv,
             b_send_cw, b_recv_cw, b_send_ccw, b_recv_ccw):
        p = lax.axis_index("i")
        g = p % N_RING
        base = p - g
        pb = (base + N_RING) % N_DEV
        right = base + (g + 1) % N_RING
        left = base + (g + N_RING - 1) % N_RING
        zp = (p + N_RING) % N_DEV
        my_off = jnp.where(p < N_RING, 0, CH)
        pa_off = CH - my_off

        bar = pltpu.get_barrier_semaphore()
        for nbr in (left, right, zp):
            pl.semaphore_signal(bar, inc=1, device_id=(nbr,),
                                device_id_type=pl.DeviceIdType.MESH)
        pl.semaphore_wait(bar, 3)

        x8[...] = x_ref[...].astype(x8.dtype)
        w8[...] = w_ref[...].astype(w8.dtype)

        def pdot(row, c0, width):
            return jnp.dot(x8[pl.ds(row, CH), :], w8[:, c0:c0 + width],
                           preferred_element_type=f32)

        def bz_push(t, half):
            if half == 0:
                idx = (g + N_RING - 1 - t) % N_RING
            else:
                idx = (g + 1 + t) % N_RING
            row = idx * CH
            rsl = pl.ds(row, CH)
            csl = pl.ds(half * BW, BW)
            bsend[rsl, csl] = pdot(
                (pb + idx) * CH, n - 2 * BW + half * BW, BW).astype(bf16)
            rd = pltpu.make_async_remote_copy(
                src_ref=bsend.at[rsl, csl],
                dst_ref=zbr.at[rsl, csl],
                send_sem=bz_send.at[t, half], recv_sem=bz_recv.at[t, half],
                device_id=(zp,), device_id_type=pl.DeviceIdType.MESH)
            rd.start()
            return rd

        def a_spartial(c, c0):
            return pdot(c * CH, c0, AW), pdot((c + N_RING) * CH, c0, AW)

        def a_rdma(comm, ss, rs, h, j, tgt):
            sl = pl.ds(j * ASW, ASW)
            return pltpu.make_async_remote_copy(
                src_ref=comm.at[h, :, sl], dst_ref=comm.at[h + 1, :, sl],
                send_sem=ss.at[h, j], recv_sem=rs.at[h + 1, j],
                device_id=(tgt,), device_id_type=pl.DeviceIdType.MESH)

        def a_accum(comm, h, j, top, bot):
            cs = j * ASW
            sl = pl.ds(cs, ASW)
            comm[h, 0:CH, sl] = (comm[h, 0:CH, sl].astype(f32)
                                 + top[:, cs:cs + ASW]).astype(bf16)
            comm[h, CH:2 * CH, sl] = (comm[h, CH:2 * CH, sl].astype(f32)
                                      + bot[:, cs:cs + ASW]).astype(bf16)

        bz = [(bz_push(0, 0), bz_push(0, 1))]

        t_, b_ = a_spartial((g + N_RING - 1) % N_RING, 0)
        comm_a_cw[0, 0:CH, :] = t_.astype(bf16)
        comm_a_cw[0, CH:2 * CH, :] = b_.astype(bf16)
        t_, b_ = a_spartial((g + 1) % N_RING, AW)
        comm_a_ccw[0, 0:CH, :] = t_.astype(bf16)
        comm_a_ccw[0, CH:2 * CH, :] = b_.astype(bf16)
        prev = []
        for j in range(ASUB):
            rd_cw = a_rdma(comm_a_cw, a_send_cw, a_recv_cw, 0, j, right)
            rd_ccw = a_rdma(comm_a_ccw, a_send_ccw, a_recv_ccw, 0, j, left)
            rd_cw.start()
            rd_ccw.start()
            prev.append((rd_cw, rd_ccw))

        bz.append((bz_push(1, 0), bz_push(1, 1)))

        for h in range(1, N_RING - 1):
            tcw, bcw = a_spartial((g + N_RING - 1 - h) % N_RING, 0)
            tccw, bccw = a_spartial((g + 1 + h) % N_RING, AW)
            cur = []
            for j in range(ASUB):
                rd_cw, rd_ccw = prev[j]
                rd_cw.wait()
                a_accum(comm_a_cw, h, j, tcw, bcw)
                nrd_cw = a_rdma(comm_a_cw, a_send_cw, a_recv_cw, h, j, right)
                nrd_cw.start()
                rd_ccw.wait()
                a_accum(comm_a_ccw, h, j, tccw, bccw)
                nrd_ccw = a_rdma(comm_a_ccw, a_send_ccw, a_recv_ccw, h, j, left)
                nrd_ccw.start()
                cur.append((nrd_cw, nrd_ccw))
            prev = cur
            bz.append((bz_push(h + 1, 0), bz_push(h + 1, 1)))

        for i in range(N_RING):
            bmine[i * CH:(i + 1) * CH, :] = pdot(
                (base + i) * CH, n - 2 * BW, 2 * BW).astype(bf16)

        def b_val(c, half):
            sl = pl.ds(half * BW, BW)
            row = pl.ds(c * CH, CH)
            return (bmine[row, sl].astype(f32) + zbr[row, sl].astype(f32))

        def b_rdma(comm, ss, rs, h, tgt):
            return pltpu.make_async_remote_copy(
                src_ref=comm.at[h], dst_ref=comm.at[h + 1],
                send_sem=ss.at[h], recv_sem=rs.at[h + 1],
                device_id=(tgt,), device_id_type=pl.DeviceIdType.MESH)

        bz[0][0].wait()
        comm_b_cw[0, :, :] = b_val((g + N_RING - 1) % N_RING, 0).astype(bf16)
        prev_b_cw = b_rdma(comm_b_cw, b_send_cw, b_recv_cw, 0, right)
        prev_b_cw.start()
        bz[0][1].wait()
        comm_b_ccw[0, :, :] = b_val((g + 1) % N_RING, 1).astype(bf16)
        prev_b_ccw = b_rdma(comm_b_ccw, b_send_ccw, b_recv_ccw, 0, left)
        prev_b_ccw.start()

        tcw, bcw = a_spartial(g, 0)
        tccw, bccw = a_spartial(g, AW)
        last = N_RING - 1
        az = []
        for j in range(ASUB):
            sl = pl.ds(j * ASW, ASW)
            rd_cw, rd_ccw = prev[j]
            rd_cw.wait()
            a_accum(comm_a_cw, last, j, tcw, bcw)
            z_cw = pltpu.make_async_remote_copy(
                src_ref=comm_a_cw.at[last, pl.ds(pa_off, CH), sl],
                dst_ref=za_cw.at[:, sl],
                send_sem=az_send_cw.at[j], recv_sem=az_recv_cw.at[j],
                device_id=(zp,), device_id_type=pl.DeviceIdType.MESH)
            z_cw.start()
            rd_ccw.wait()
            a_accum(comm_a_ccw, last, j, tccw, bccw)
            z_ccw = pltpu.make_async_remote_copy(
                src_ref=comm_a_ccw.at[last, pl.ds(pa_off, CH), sl],
                dst_ref=za_ccw.at[:, sl],
                send_sem=az_send_ccw.at[j], recv_sem=az_recv_ccw.at[j],
                device_id=(zp,), device_id_type=pl.DeviceIdType.MESH)
            z_ccw.start()
            az.append((z_cw, z_ccw))

        for h in range(1, N_RING - 1):
            bz[h][0].wait()
            prev_b_cw.wait()
            comm_b_cw[h, :, :] = (
                comm_b_cw[h, :, :].astype(f32)
                + b_val((g + N_RING - 1 - h) % N_RING, 0)).astype(bf16)
            prev_b_cw = b_rdma(comm_b_cw, b_send_cw, b_recv_cw, h, right)
            prev_b_cw.start()
            bz[h][1].wait()
            prev_b_ccw.wait()
            comm_b_ccw[h, :, :] = (
                comm_b_ccw[h, :, :].astype(f32)
                + b_val((g + 1 + h) % N_RING, 1)).astype(bf16)
            prev_b_ccw = b_rdma(comm_b_ccw, b_send_ccw, b_recv_ccw, h, left)
            prev_b_ccw.start()

        s = sx_ref[0] * sw_ref[0]

        bz[N_RING - 1][0].wait()
        bz[N_RING - 1][1].wait()
        prev_b_cw.wait()
        prev_b_ccw.wait()
        out_ref[:, n - 2 * BW:n - BW] = jnp.maximum(
            (comm_b_cw[last, :, :].astype(f32) + b_val(g, 0)) * s, 0.0)
        out_ref[:, n - BW:n] = jnp.maximum(
            (comm_b_ccw[last, :, :].astype(f32) + b_val(g, 1)) * s, 0.0)

        for z_cw, z_ccw in az:
            z_cw.wait()
            z_ccw.wait()
        acc_cw = (comm_a_cw[last, pl.ds(my_off, CH), :].astype(f32)
                  + za_cw[...].astype(f32))
        acc_ccw = (comm_a_ccw[last, pl.ds(my_off, CH), :].astype(f32)
                   + za_ccw[...].astype(f32))
        out_ref[:, 0:AW] = jnp.maximum(acc_cw * s, 0.0)
        out_ref[:, AW:2 * AW] = jnp.maximum(acc_ccw * s, 0.0)

    return pl.pallas_call(
        body,
        out_shape=jax.ShapeDtypeStruct((CH, n), f32),
        in_specs=[
            pl.BlockSpec(memory_space=pltpu.VMEM),
            pl.BlockSpec(memory_space=pltpu.VMEM),
            pl.BlockSpec(memory_space=pltpu.SMEM),
            pl.BlockSpec(memory_space=pltpu.SMEM),
        ],
        out_specs=pl.BlockSpec(memory_space=pltpu.VMEM),
        scratch_shapes=[
            pltpu.VMEM((m, k), jnp.float8_e4m3fn),
            pltpu.VMEM((k, n), jnp.float8_e5m2),
            pltpu.VMEM((N_RING, 2 * CH, AW), bf16),
            pltpu.VMEM((N_RING, 2 * CH, AW), bf16),
            pltpu.VMEM((CH, AW), bf16),
            pltpu.VMEM((CH, AW), bf16),
            pltpu.VMEM((N_RING * CH, 2 * BW), bf16),
            pltpu.VMEM((N_RING * CH, 2 * BW), bf16),
            pltpu.VMEM((N_RING * CH, 2 * BW), bf16),
            pltpu.VMEM((N_RING, CH, BW), bf16),
            pltpu.VMEM((N_RING, CH, BW), bf16),
            pltpu.SemaphoreType.DMA((N_RING, ASUB)),
            pltpu.SemaphoreType.DMA((N_RING, ASUB)),
            pltpu.SemaphoreType.DMA((N_RING, ASUB)),
            pltpu.SemaphoreType.DMA((N_RING, ASUB)),
            pltpu.SemaphoreType.DMA((ASUB,)),
            pltpu.SemaphoreType.DMA((ASUB,)),
            pltpu.SemaphoreType.DMA((ASUB,)),
            pltpu.SemaphoreType.DMA((ASUB,)),
            pltpu.SemaphoreType.DMA((N_RING, 2)),
            pltpu.SemaphoreType.DMA((N_RING, 2)),
            pltpu.SemaphoreType.DMA((N_RING,)),
            pltpu.SemaphoreType.DMA((N_RING,)),
            pltpu.SemaphoreType.DMA((N_RING,)),
            pltpu.SemaphoreType.DMA((N_RING,)),
        ],
        compiler_params=pltpu.CompilerParams(collective_id=0),
    )(x, w_mat, scale_x, scale_w)


# device time: 62692 ns/iter; 1.5415x vs baseline; 1.1306x over previous
import jax
import jax.numpy as jnp
from jax import lax
from jax.experimental import pallas as pl
from jax.experimental.pallas import tpu as pltpu

N_DEV = 8
N_RING = 4
CH = 512
AW = 256
BW = 768
ASUB = 2
ASW = AW // ASUB


def kernel(x, w_mat, scale_x, scale_w):
    m, k = x.shape
    _, n = w_mat.shape
    f32 = jnp.float32
    bf16 = jnp.bfloat16

    def body(x_ref, w_ref, sx_ref, sw_ref, out_ref,
             x8, w8,
             comm_a_cw, comm_a_ccw, za_cw, za_ccw,
             bsend, bmine, zbr, comm_b_cw, comm_b_ccw,
             a_send_cw, a_recv_cw, a_send_ccw, a_recv_ccw,
             az_send_cw, az_recv_cw, az_send_ccw, az_recv_ccw,
             bz_send, bz_recv,
             b_send_cw, b_recv_cw, b_send_ccw, b_recv_ccw):
        p = lax.axis_index("i")
        g = p % N_RING
        base = p - g
        pb = (base + N_RING) % N_DEV
        right = base + (g + 1) % N_RING
        left = base + (g + N_RING - 1) % N_RING
        zp = (p + N_RING) % N_DEV
        my_off = jnp.where(p < N_RING, 0, CH)
        pa_off = CH - my_off

        bar = pltpu.get_barrier_semaphore()
        for nbr in (left, right, zp):
            pl.semaphore_signal(bar, inc=1, device_id=(nbr,),
                                device_id_type=pl.DeviceIdType.MESH)
        pl.semaphore_wait(bar, 3)

        x8[...] = x_ref[...].astype(x8.dtype)
        w8[...] = w_ref[...].astype(w8.dtype)

        def pdot(row, c0, width):
            return jnp.dot(x8[pl.ds(row, CH), :], w8[:, c0:c0 + width],
                           preferred_element_type=f32)

        def bz_push(t, half):
            if half == 0:
                idx = (g + N_RING - 1 - t) % N_RING
            else:
                idx = (g + 1 + t) % N_RING
            row = idx * CH
            rsl = pl.ds(row, CH)
            csl = pl.ds(half * BW, BW)
            bsend[rsl, csl] = pdot(
                (pb + idx) * CH, n - 2 * BW + half * BW, BW).astype(
                    jnp.float8_e4m3fn)
            rd = pltpu.make_async_remote_copy(
                src_ref=bsend.at[rsl, csl],
                dst_ref=zbr.at[rsl, csl],
                send_sem=bz_send.at[t, half], recv_sem=bz_recv.at[t, half],
                device_id=(zp,), device_id_type=pl.DeviceIdType.MESH)
            rd.start()
            return rd

        def a_spartial(c, c0):
            return pdot(c * CH, c0, AW), pdot((c + N_RING) * CH, c0, AW)

        def a_rdma(comm, ss, rs, h, j, tgt):
            sl = pl.ds(j * ASW, ASW)
            return pltpu.make_async_remote_copy(
                src_ref=comm.at[h, :, sl], dst_ref=comm.at[h + 1, :, sl],
                send_sem=ss.at[h, j], recv_sem=rs.at[h + 1, j],
                device_id=(tgt,), device_id_type=pl.DeviceIdType.MESH)

        def a_accum(comm, h, j, top, bot):
            cs = j * ASW
            sl = pl.ds(cs, ASW)
            comm[h, 0:CH, sl] = (comm[h, 0:CH, sl].astype(f32)
                                 + top[:, cs:cs + ASW]).astype(bf16)
            comm[h, CH:2 * CH, sl] = (comm[h, CH:2 * CH, sl].astype(f32)
                                      + bot[:, cs:cs + ASW]).astype(bf16)

        bz = [(bz_push(0, 0), bz_push(0, 1))]

        t_, b_ = a_spartial((g + N_RING - 1) % N_RING, 0)
        comm_a_cw[0, 0:CH, :] = t_.astype(bf16)
        comm_a_cw[0, CH:2 * CH, :] = b_.astype(bf16)
        t_, b_ = a_spartial((g + 1) % N_RING, AW)
        comm_a_ccw[0, 0:CH, :] = t_.astype(bf16)
        comm_a_ccw[0, CH:2 * CH, :] = b_.astype(bf16)
        prev = []
        for j in range(ASUB):
            rd_cw = a_rdma(comm_a_cw, a_send_cw, a_recv_cw, 0, j, right)
            rd_ccw = a_rdma(comm_a_ccw, a_send_ccw, a_recv_ccw, 0, j, left)
            rd_cw.start()
            rd_ccw.start()
            prev.append((rd_cw, rd_ccw))

        bz.append((bz_push(1, 0), bz_push(1, 1)))

        for h in range(1, N_RING - 1):
            tcw, bcw = a_spartial((g + N_RING - 1 - h) % N_RING, 0)
            tccw, bccw = a_spartial((g + 1 + h) % N_RING, AW)
            cur = []
            for j in range(ASUB):
                rd_cw, rd_ccw = prev[j]
                rd_cw.wait()
                a_accum(comm_a_cw, h, j, tcw, bcw)
                nrd_cw = a_rdma(comm_a_cw, a_send_cw, a_recv_cw, h, j, right)
                nrd_cw.start()
                rd_ccw.wait()
                a_accum(comm_a_ccw, h, j, tccw, bccw)
                nrd_ccw = a_rdma(comm_a_ccw, a_send_ccw, a_recv_ccw, h, j, left)
                nrd_ccw.start()
                cur.append((nrd_cw, nrd_ccw))
            prev = cur
            bz.append((bz_push(h + 1, 0), bz_push(h + 1, 1)))

        for i in range(N_RING):
            bmine[i * CH:(i + 1) * CH, :] = pdot(
                (base + i) * CH, n - 2 * BW, 2 * BW).astype(bf16)

        def b_val(c, half):
            sl = pl.ds(half * BW, BW)
            row = pl.ds(c * CH, CH)
            return (bmine[row, sl].astype(f32) + zbr[row, sl].astype(f32))

        def b_rdma(comm, ss, rs, h, tgt):
            return pltpu.make_async_remote_copy(
                src_ref=comm.at[h], dst_ref=comm.at[h + 1],
                send_sem=ss.at[h], recv_sem=rs.at[h + 1],
                device_id=(tgt,), device_id_type=pl.DeviceIdType.MESH)

        bz[0][0].wait()
        comm_b_cw[0, :, :] = b_val((g + N_RING - 1) % N_RING, 0).astype(bf16)
        prev_b_cw = b_rdma(comm_b_cw, b_send_cw, b_recv_cw, 0, right)
        prev_b_cw.start()
        bz[0][1].wait()
        comm_b_ccw[0, :, :] = b_val((g + 1) % N_RING, 1).astype(bf16)
        prev_b_ccw = b_rdma(comm_b_ccw, b_send_ccw, b_recv_ccw, 0, left)
        prev_b_ccw.start()

        tcw, bcw = a_spartial(g, 0)
        tccw, bccw = a_spartial(g, AW)
        last = N_RING - 1
        az = []
        for j in range(ASUB):
            sl = pl.ds(j * ASW, ASW)
            rd_cw, rd_ccw = prev[j]
            rd_cw.wait()
            a_accum(comm_a_cw, last, j, tcw, bcw)
            z_cw = pltpu.make_async_remote_copy(
                src_ref=comm_a_cw.at[last, pl.ds(pa_off, CH), sl],
                dst_ref=za_cw.at[:, sl],
                send_sem=az_send_cw.at[j], recv_sem=az_recv_cw.at[j],
                device_id=(zp,), device_id_type=pl.DeviceIdType.MESH)
            z_cw.start()
            rd_ccw.wait()
            a_accum(comm_a_ccw, last, j, tccw, bccw)
            z_ccw = pltpu.make_async_remote_copy(
                src_ref=comm_a_ccw.at[last, pl.ds(pa_off, CH), sl],
                dst_ref=za_ccw.at[:, sl],
                send_sem=az_send_ccw.at[j], recv_sem=az_recv_ccw.at[j],
                device_id=(zp,), device_id_type=pl.DeviceIdType.MESH)
            z_ccw.start()
            az.append((z_cw, z_ccw))

        for h in range(1, N_RING - 1):
            bz[h][0].wait()
            prev_b_cw.wait()
            comm_b_cw[h, :, :] = (
                comm_b_cw[h, :, :].astype(f32)
                + b_val((g + N_RING - 1 - h) % N_RING, 0)).astype(bf16)
            prev_b_cw = b_rdma(comm_b_cw, b_send_cw, b_recv_cw, h, right)
            prev_b_cw.start()
            bz[h][1].wait()
            prev_b_ccw.wait()
            comm_b_ccw[h, :, :] = (
                comm_b_ccw[h, :, :].astype(f32)
                + b_val((g + 1 + h) % N_RING, 1)).astype(bf16)
            prev_b_ccw = b_rdma(comm_b_ccw, b_send_ccw, b_recv_ccw, h, left)
            prev_b_ccw.start()

        s = sx_ref[0] * sw_ref[0]

        bz[N_RING - 1][0].wait()
        bz[N_RING - 1][1].wait()
        prev_b_cw.wait()
        prev_b_ccw.wait()
        out_ref[:, n - 2 * BW:n - BW] = jnp.maximum(
            (comm_b_cw[last, :, :].astype(f32) + b_val(g, 0)) * s, 0.0)
        out_ref[:, n - BW:n] = jnp.maximum(
            (comm_b_ccw[last, :, :].astype(f32) + b_val(g, 1)) * s, 0.0)

        for z_cw, z_ccw in az:
            z_cw.wait()
            z_ccw.wait()
        acc_cw = (comm_a_cw[last, pl.ds(my_off, CH), :].astype(f32)
                  + za_cw[...].astype(f32))
        acc_ccw = (comm_a_ccw[last, pl.ds(my_off, CH), :].astype(f32)
                   + za_ccw[...].astype(f32))
        out_ref[:, 0:AW] = jnp.maximum(acc_cw * s, 0.0)
        out_ref[:, AW:2 * AW] = jnp.maximum(acc_ccw * s, 0.0)

    return pl.pallas_call(
        body,
        out_shape=jax.ShapeDtypeStruct((CH, n), f32),
        in_specs=[
            pl.BlockSpec(memory_space=pltpu.VMEM),
            pl.BlockSpec(memory_space=pltpu.VMEM),
            pl.BlockSpec(memory_space=pltpu.SMEM),
            pl.BlockSpec(memory_space=pltpu.SMEM),
        ],
        out_specs=pl.BlockSpec(memory_space=pltpu.VMEM),
        scratch_shapes=[
            pltpu.VMEM((m, k), jnp.float8_e4m3fn),
            pltpu.VMEM((k, n), jnp.float8_e5m2),
            pltpu.VMEM((N_RING, 2 * CH, AW), bf16),
            pltpu.VMEM((N_RING, 2 * CH, AW), bf16),
            pltpu.VMEM((CH, AW), bf16),
            pltpu.VMEM((CH, AW), bf16),
            pltpu.VMEM((N_RING * CH, 2 * BW), jnp.float8_e4m3fn),
            pltpu.VMEM((N_RING * CH, 2 * BW), bf16),
            pltpu.VMEM((N_RING * CH, 2 * BW), jnp.float8_e4m3fn),
            pltpu.VMEM((N_RING, CH, BW), bf16),
            pltpu.VMEM((N_RING, CH, BW), bf16),
            pltpu.SemaphoreType.DMA((N_RING, ASUB)),
            pltpu.SemaphoreType.DMA((N_RING, ASUB)),
            pltpu.SemaphoreType.DMA((N_RING, ASUB)),
            pltpu.SemaphoreType.DMA((N_RING, ASUB)),
            pltpu.SemaphoreType.DMA((ASUB,)),
            pltpu.SemaphoreType.DMA((ASUB,)),
            pltpu.SemaphoreType.DMA((ASUB,)),
            pltpu.SemaphoreType.DMA((ASUB,)),
            pltpu.SemaphoreType.DMA((N_RING, 2)),
            pltpu.SemaphoreType.DMA((N_RING, 2)),
            pltpu.SemaphoreType.DMA((N_RING,)),
            pltpu.SemaphoreType.DMA((N_RING,)),
            pltpu.SemaphoreType.DMA((N_RING,)),
            pltpu.SemaphoreType.DMA((N_RING,)),
        ],
        compiler_params=pltpu.CompilerParams(collective_id=0),
    )(x, w_mat, scale_x, scale_w)


# device time: 61634 ns/iter; 1.5680x vs baseline; 1.0172x over previous
import jax
import jax.numpy as jnp
from jax import lax
from jax.experimental import pallas as pl
from jax.experimental.pallas import tpu as pltpu

N_DEV = 8
N_RING = 4
CH = 512
AW = 256
BW = 768
ASUB = 2
ASW = AW // ASUB


def kernel(x, w_mat, scale_x, scale_w):
    m, k = x.shape
    _, n = w_mat.shape
    f32 = jnp.float32
    bf16 = jnp.bfloat16

    def body(x_ref, w_ref, sx_ref, sw_ref, out_ref,
             x8, w8,
             comm_a_cw, comm_a_ccw, za_cw, za_ccw,
             bsend, bmine, zbr, comm_b_cw, comm_b_ccw,
             a_send_cw, a_recv_cw, a_send_ccw, a_recv_ccw,
             az_send_cw, az_recv_cw, az_send_ccw, az_recv_ccw,
             bz_send, bz_recv,
             b_send_cw, b_recv_cw, b_send_ccw, b_recv_ccw):
        p = lax.axis_index("i")
        g = p % N_RING
        base = p - g
        pb = (base + N_RING) % N_DEV
        right = base + (g + 1) % N_RING
        left = base + (g + N_RING - 1) % N_RING
        zp = (p + N_RING) % N_DEV
        my_off = jnp.where(p < N_RING, 0, CH)
        pa_off = CH - my_off

        bar = pltpu.get_barrier_semaphore()
        for nbr in (left, right, zp):
            pl.semaphore_signal(bar, inc=1, device_id=(nbr,),
                                device_id_type=pl.DeviceIdType.MESH)
        pl.semaphore_wait(bar, 3)

        x8[...] = x_ref[...].astype(x8.dtype)
        w8[:, 0:2 * AW] = w_ref[:, 0:2 * AW].astype(w8.dtype)

        def pdot(row, c0, width):
            return jnp.dot(x8[pl.ds(row, CH), :], w8[:, c0:c0 + width],
                           preferred_element_type=f32)

        def bz_push(t, half):
            if half == 0:
                idx = (g + N_RING - 1 - t) % N_RING
            else:
                idx = (g + 1 + t) % N_RING
            row = idx * CH
            rsl = pl.ds(row, CH)
            csl = pl.ds(half * BW, BW)
            bsend[rsl, csl] = pdot(
                (pb + idx) * CH, n - 2 * BW + half * BW, BW).astype(
                    jnp.float8_e4m3fn)
            rd = pltpu.make_async_remote_copy(
                src_ref=bsend.at[rsl, csl],
                dst_ref=zbr.at[rsl, csl],
                send_sem=bz_send.at[t, half], recv_sem=bz_recv.at[t, half],
                device_id=(zp,), device_id_type=pl.DeviceIdType.MESH)
            rd.start()
            return rd

        def a_spartial(c, c0):
            return pdot(c * CH, c0, AW), pdot((c + N_RING) * CH, c0, AW)

        def a_rdma(comm, ss, rs, h, j, tgt):
            sl = pl.ds(j * ASW, ASW)
            return pltpu.make_async_remote_copy(
                src_ref=comm.at[h, :, sl], dst_ref=comm.at[h + 1, :, sl],
                send_sem=ss.at[h, j], recv_sem=rs.at[h + 1, j],
                device_id=(tgt,), device_id_type=pl.DeviceIdType.MESH)

        def a_accum(comm, h, j, top, bot):
            cs = j * ASW
            sl = pl.ds(cs, ASW)
            comm[h, 0:CH, sl] = (comm[h, 0:CH, sl].astype(f32)
                                 + top[:, cs:cs + ASW]).astype(bf16)
            comm[h, CH:2 * CH, sl] = (comm[h, CH:2 * CH, sl].astype(f32)
                                      + bot[:, cs:cs + ASW]).astype(bf16)

        t_, b_ = a_spartial((g + N_RING - 1) % N_RING, 0)
        comm_a_cw[0, 0:CH, :] = t_.astype(bf16)
        comm_a_cw[0, CH:2 * CH, :] = b_.astype(bf16)
        t_, b_ = a_spartial((g + 1) % N_RING, AW)
        comm_a_ccw[0, 0:CH, :] = t_.astype(bf16)
        comm_a_ccw[0, CH:2 * CH, :] = b_.astype(bf16)
        prev = []
        for j in range(ASUB):
            rd_cw = a_rdma(comm_a_cw, a_send_cw, a_recv_cw, 0, j, right)
            rd_ccw = a_rdma(comm_a_ccw, a_send_ccw, a_recv_ccw, 0, j, left)
            rd_cw.start()
            rd_ccw.start()
            prev.append((rd_cw, rd_ccw))

        w8[:, 2 * AW:] = w_ref[:, 2 * AW:].astype(w8.dtype)
        bz = [(bz_push(0, 0), bz_push(0, 1))]
        bz.append((bz_push(1, 0), bz_push(1, 1)))

        for h in range(1, N_RING - 1):
            tcw, bcw = a_spartial((g + N_RING - 1 - h) % N_RING, 0)
            tccw, bccw = a_spartial((g + 1 + h) % N_RING, AW)
            cur = []
            for j in range(ASUB):
                rd_cw, rd_ccw = prev[j]
                rd_cw.wait()
                a_accum(comm_a_cw, h, j, tcw, bcw)
                nrd_cw = a_rdma(comm_a_cw, a_send_cw, a_recv_cw, h, j, right)
                nrd_cw.start()
                rd_ccw.wait()
                a_accum(comm_a_ccw, h, j, tccw, bccw)
                nrd_ccw = a_rdma(comm_a_ccw, a_send_ccw, a_recv_ccw, h, j, left)
                nrd_ccw.start()
                cur.append((nrd_cw, nrd_ccw))
            prev = cur
            bz.append((bz_push(h + 1, 0), bz_push(h + 1, 1)))

        for i in range(N_RING):
            bmine[i * CH:(i + 1) * CH, :] = pdot(
                (base + i) * CH, n - 2 * BW, 2 * BW).astype(bf16)

        def b_val(c, half):
            sl = pl.ds(half * BW, BW)
            row = pl.ds(c * CH, CH)
            return (bmine[row, sl].astype(f32) + zbr[row, sl].astype(f32))

        def b_rdma(comm, ss, rs, h, tgt):
            return pltpu.make_async_remote_copy(
                src_ref=comm.at[h], dst_ref=comm.at[h + 1],
                send_sem=ss.at[h], recv_sem=rs.at[h + 1],
                device_id=(tgt,), device_id_type=pl.DeviceIdType.MESH)

        bz[0][0].wait()
        comm_b_cw[0, :, :] = b_val((g + N_RING - 1) % N_RING, 0).astype(bf16)
        prev_b_cw = b_rdma(comm_b_cw, b_send_cw, b_recv_cw, 0, right)
        prev_b_cw.start()
        bz[0][1].wait()
        comm_b_ccw[0, :, :] = b_val((g + 1) % N_RING, 1).astype(bf16)
        prev_b_ccw = b_rdma(comm_b_ccw, b_send_ccw, b_recv_ccw, 0, left)
        prev_b_ccw.start()

        tcw, bcw = a_spartial(g, 0)
        tccw, bccw = a_spartial(g, AW)
        last = N_RING - 1
        az = []
        for j in range(ASUB):
            sl = pl.ds(j * ASW, ASW)
            rd_cw, rd_ccw = prev[j]
            rd_cw.wait()
            a_accum(comm_a_cw, last, j, tcw, bcw)
            z_cw = pltpu.make_async_remote_copy(
                src_ref=comm_a_cw.at[last, pl.ds(pa_off, CH), sl],
                dst_ref=za_cw.at[:, sl],
                send_sem=az_send_cw.at[j], recv_sem=az_recv_cw.at[j],
                device_id=(zp,), device_id_type=pl.DeviceIdType.MESH)
            z_cw.start()
            rd_ccw.wait()
            a_accum(comm_a_ccw, last, j, tccw, bccw)
            z_ccw = pltpu.make_async_remote_copy(
                src_ref=comm_a_ccw.at[last, pl.ds(pa_off, CH), sl],
                dst_ref=za_ccw.at[:, sl],
                send_sem=az_send_ccw.at[j], recv_sem=az_recv_ccw.at[j],
                device_id=(zp,), device_id_type=pl.DeviceIdType.MESH)
            z_ccw.start()
            az.append((z_cw, z_ccw))

        for h in range(1, N_RING - 1):
            bz[h][0].wait()
            prev_b_cw.wait()
            comm_b_cw[h, :, :] = (
                comm_b_cw[h, :, :].astype(f32)
                + b_val((g + N_RING - 1 - h) % N_RING, 0)).astype(bf16)
            prev_b_cw = b_rdma(comm_b_cw, b_send_cw, b_recv_cw, h, right)
            prev_b_cw.start()
            bz[h][1].wait()
            prev_b_ccw.wait()
            comm_b_ccw[h, :, :] = (
                comm_b_ccw[h, :, :].astype(f32)
                + b_val((g + 1 + h) % N_RING, 1)).astype(bf16)
            prev_b_ccw = b_rdma(comm_b_ccw, b_send_ccw, b_recv_ccw, h, left)
            prev_b_ccw.start()

        s = sx_ref[0] * sw_ref[0]

        bz[N_RING - 1][0].wait()
        bz[N_RING - 1][1].wait()
        prev_b_cw.wait()
        prev_b_ccw.wait()
        out_ref[:, n - 2 * BW:n - BW] = jnp.maximum(
            (comm_b_cw[last, :, :].astype(f32) + b_val(g, 0)) * s, 0.0)
        out_ref[:, n - BW:n] = jnp.maximum(
            (comm_b_ccw[last, :, :].astype(f32) + b_val(g, 1)) * s, 0.0)

        for z_cw, z_ccw in az:
            z_cw.wait()
            z_ccw.wait()
        acc_cw = (comm_a_cw[last, pl.ds(my_off, CH), :].astype(f32)
                  + za_cw[...].astype(f32))
        acc_ccw = (comm_a_ccw[last, pl.ds(my_off, CH), :].astype(f32)
                   + za_ccw[...].astype(f32))
        out_ref[:, 0:AW] = jnp.maximum(acc_cw * s, 0.0)
        out_ref[:, AW:2 * AW] = jnp.maximum(acc_ccw * s, 0.0)

    return pl.pallas_call(
        body,
        out_shape=jax.ShapeDtypeStruct((CH, n), f32),
        in_specs=[
            pl.BlockSpec(memory_space=pltpu.VMEM),
            pl.BlockSpec(memory_space=pltpu.VMEM),
            pl.BlockSpec(memory_space=pltpu.SMEM),
            pl.BlockSpec(memory_space=pltpu.SMEM),
        ],
        out_specs=pl.BlockSpec(memory_space=pltpu.VMEM),
        scratch_shapes=[
            pltpu.VMEM((m, k), jnp.float8_e4m3fn),
            pltpu.VMEM((k, n), jnp.float8_e5m2),
            pltpu.VMEM((N_RING, 2 * CH, AW), bf16),
            pltpu.VMEM((N_RING, 2 * CH, AW), bf16),
            pltpu.VMEM((CH, AW), bf16),
            pltpu.VMEM((CH, AW), bf16),
            pltpu.VMEM((N_RING * CH, 2 * BW), jnp.float8_e4m3fn),
            pltpu.VMEM((N_RING * CH, 2 * BW), bf16),
            pltpu.VMEM((N_RING * CH, 2 * BW), jnp.float8_e4m3fn),
            pltpu.VMEM((N_RING, CH, BW), bf16),
            pltpu.VMEM((N_RING, CH, BW), bf16),
            pltpu.SemaphoreType.DMA((N_RING, ASUB)),
            pltpu.SemaphoreType.DMA((N_RING, ASUB)),
            pltpu.SemaphoreType.DMA((N_RING, ASUB)),
            pltpu.SemaphoreType.DMA((N_RING, ASUB)),
            pltpu.SemaphoreType.DMA((ASUB,)),
            pltpu.SemaphoreType.DMA((ASUB,)),
            pltpu.SemaphoreType.DMA((ASUB,)),
            pltpu.SemaphoreType.DMA((ASUB,)),
            pltpu.SemaphoreType.DMA((N_RING, 2)),
            pltpu.SemaphoreType.DMA((N_RING, 2)),
            pltpu.SemaphoreType.DMA((N_RING,)),
            pltpu.SemaphoreType.DMA((N_RING,)),
            pltpu.SemaphoreType.DMA((N_RING,)),
            pltpu.SemaphoreType.DMA((N_RING,)),
        ],
        compiler_params=pltpu.CompilerParams(collective_id=0),
    )(x, w_mat, scale_x, scale_w)
